# baseline (speedup 1.0000x reference)
import jax
import jax.numpy as jnp
from jax import lax
from jax.experimental import pallas as pl
from jax.experimental.pallas import tpu as pltpu

_TEMP = 0.7
_ENTROPY_TRIG = 2.2
_EPS = 1e-5
_E = 16
_VT = 3200


def _ln(x, w, b):
    mu = jnp.mean(x, axis=-1, keepdims=True)
    var = jnp.mean((x - mu) ** 2, axis=-1, keepdims=True)
    return (x - mu) / jnp.sqrt(var + _EPS) * w + b


_EMB_TILE = 256


def _embed_body(ids_ref, tok_hbm, pos_ref, out_ref, sem):
    i = pl.program_id(0)
    base = i * _EMB_TILE
    copies = []
    for mi in range(_EMB_TILE):
        c = pltpu.make_async_copy(
            tok_hbm.at[pl.ds(ids_ref[base + mi], 1), :], out_ref.at[mi], sem)
        c.start()
        copies.append(c)
    for c in copies:
        c.wait()
    out_ref[...] = out_ref[...] + pos_ref[...]


def _embed(ids, tok, pos):
    n = ids.shape[0]
    t = pos.shape[0]
    d = tok.shape[1]
    pos3 = pos.reshape(t, 1, d)
    n_tiles = n // _EMB_TILE
    pos_tiles = t // _EMB_TILE
    out = pl.pallas_call(
        _embed_body,
        out_shape=jax.ShapeDtypeStruct((n, 1, d), jnp.float32),
        grid_spec=pltpu.PrefetchScalarGridSpec(
            num_scalar_prefetch=1,
            grid=(n_tiles,),
            in_specs=[
                pl.BlockSpec(memory_space=pl.ANY),
                pl.BlockSpec((_EMB_TILE, 1, d),
                             lambda i, ids_r: (i % pos_tiles, 0, 0)),
            ],
            out_specs=pl.BlockSpec((_EMB_TILE, 1, d),
                                   lambda i, ids_r: (i, 0, 0)),
            scratch_shapes=[pltpu.SemaphoreType.DMA],
        ),
        compiler_params=pltpu.CompilerParams(
            dimension_semantics=("arbitrary",)),
        name="embed_gather",
    )(ids, tok, pos3)
    return out.reshape(n, d)


_EH = _E // 2


def _moe_body(x_ref, lnw_ref, lnb_ref, rw_ref, rb_ref,
              w1_ref, b1_ref, w2_ref, b2_ref,
              moe_ref, hn_ref, wf_ref):
    c = pl.program_id(0)
    e = pl.program_id(1)

    @pl.when(e == 0)
    def _():
        x = x_ref[...]
        hn = _ln(x, lnw_ref[...], lnb_ref[...])
        hn_ref[...] = hn
        logits = jnp.dot(hn, rw_ref[...],
                         preferred_element_type=jnp.float32) + rb_ref[...]
        g = jax.nn.softmax(logits / _TEMP, axis=-1)
        iota = lax.broadcasted_iota(jnp.int32, g.shape, 1)
        m1 = jnp.max(g, axis=-1, keepdims=True)
        a1 = jnp.min(jnp.where(g == m1, iota, _E), axis=-1, keepdims=True)
        g2 = jnp.where(iota == a1, -jnp.inf, g)
        m2 = jnp.max(g2, axis=-1, keepdims=True)
        a2 = jnp.min(jnp.where(g2 == m2, iota, _E), axis=-1, keepdims=True)
        wf_ref[...] = (jnp.where(iota == a1, m1, 0.0)
                       + jnp.where(iota == a2, m2, 0.0))
        moe_ref[...] = jnp.zeros_like(moe_ref)

    n = hn_ref.shape[0]
    iota = lax.broadcasted_iota(jnp.int32, wf_ref.shape, 1)
    wcol_all = jnp.sum(
        wf_ref[...] * (iota == (c * _EH + e)).astype(jnp.float32),
        axis=-1, keepdims=True)
    for sub in range(2):
        lo, hi = sub * (n // 2), (sub + 1) * (n // 2)
        hn = hn_ref[lo:hi, :]
        mid = jax.nn.silu(
            jnp.dot(hn, w1_ref[0], preferred_element_type=jnp.float32)
            + b1_ref[0])
        eo = (jnp.dot(mid, w2_ref[0], preferred_element_type=jnp.float32)
              + b2_ref[0])
        moe_ref[0, lo:hi, :] = (moe_ref[0, lo:hi, :]
                                + wcol_all[lo:hi, :] * eo)


def _moe(x, lp):
    n, d = x.shape
    h4 = lp["e_w1"].shape[2]
    return pl.pallas_call(
        _moe_body,
        out_shape=jax.ShapeDtypeStruct((2, n, d), jnp.float32),
        grid=(2, _EH),
        in_specs=[
            pl.BlockSpec((n, d), lambda c, e: (0, 0)),
            pl.BlockSpec((1, d), lambda c, e: (0, 0)),
            pl.BlockSpec((1, d), lambda c, e: (0, 0)),
            pl.BlockSpec((d, _E), lambda c, e: (0, 0)),
            pl.BlockSpec((1, _E), lambda c, e: (0, 0)),
            pl.BlockSpec((1, d, h4), lambda c, e: (c * _EH + e, 0, 0)),
            pl.BlockSpec((1, 1, h4), lambda c, e: (c * _EH + e, 0, 0)),
            pl.BlockSpec((1, h4, d), lambda c, e: (c * _EH + e, 0, 0)),
            pl.BlockSpec((1, 1, d), lambda c, e: (c * _EH + e, 0, 0)),
        ],
        out_specs=pl.BlockSpec((1, n, d), lambda c, e: (c, 0, 0)),
        scratch_shapes=[
            pltpu.VMEM((n, d), jnp.float32),
            pltpu.VMEM((n, _E), jnp.float32),
        ],
        compiler_params=pltpu.CompilerParams(
            dimension_semantics=("parallel", "arbitrary"),
            vmem_limit_bytes=50 * 1024 * 1024,
        ),
        name="moe_experts",
    )(x, lp["ln_in_w"].reshape(1, d), lp["ln_in_b"].reshape(1, d),
      lp["router_w"], lp["router_b"].reshape(1, _E),
      lp["e_w1"], lp["e_b1"].reshape(_E, 1, h4),
      lp["e_w2"], lp["e_b2"].reshape(_E, 1, d))


def _ffn_body(x_ref, moe_ref, lnw_ref, lnb_ref,
              sw1_ref, sb1_ref, sw2_ref, sb2_ref,
              flnw_ref, flnb_ref, fw1_ref, fb1_ref, fw2_ref, fb2_ref,
              out_ref):
    x = x_ref[...]
    hn = _ln(x, lnw_ref[...], lnb_ref[...])
    shared = (jnp.dot(
        jax.nn.silu(jnp.dot(hn, sw1_ref[...],
                            preferred_element_type=jnp.float32)
                    + sb1_ref[...]),
        sw2_ref[...], preferred_element_type=jnp.float32)
        + sb2_ref[...]) * 0.25
    moe = moe_ref[0] + moe_ref[1] + shared
    fh = _ln(moe, flnw_ref[...], flnb_ref[...])
    ff = (jnp.dot(
        jax.nn.silu(jnp.dot(fh, fw1_ref[...],
                            preferred_element_type=jnp.float32)
                    + fb1_ref[...]),
        fw2_ref[...], preferred_element_type=jnp.float32)
        + fb2_ref[...])
    out_ref[...] = x + moe + ff


def _ffn(x, moe, lp):
    n, d = x.shape
    d2 = lp["s_w1"].shape[1]
    h4 = lp["ff_w1"].shape[1]
    tt = n // 4
    return pl.pallas_call(
        _ffn_body,
        out_shape=jax.ShapeDtypeStruct((n, d), jnp.float32),
        grid=(4,),
        in_specs=[
            pl.BlockSpec((tt, d), lambda t: (t, 0)),
            pl.BlockSpec((2, tt, d), lambda t: (0, t, 0)),
            pl.BlockSpec((1, d), lambda t: (0, 0)),
            pl.BlockSpec((1, d), lambda t: (0, 0)),
            pl.BlockSpec((d, d2), lambda t: (0, 0)),
            pl.BlockSpec((1, d2), lambda t: (0, 0)),
            pl.BlockSpec((d2, d), lambda t: (0, 0)),
            pl.BlockSpec((1, d), lambda t: (0, 0)),
            pl.BlockSpec((1, d), lambda t: (0, 0)),
            pl.BlockSpec((1, d), lambda t: (0, 0)),
            pl.BlockSpec((d, h4), lambda t: (0, 0)),
            pl.BlockSpec((1, h4), lambda t: (0, 0)),
            pl.BlockSpec((h4, d), lambda t: (0, 0)),
            pl.BlockSpec((1, d), lambda t: (0, 0)),
        ],
        out_specs=pl.BlockSpec((tt, d), lambda t: (t, 0)),
        compiler_params=pltpu.CompilerParams(
            dimension_semantics=("parallel",),
            vmem_limit_bytes=50 * 1024 * 1024,
        ),
        name="shared_ffn",
    )(x, moe, lp["ln_in_w"].reshape(1, d), lp["ln_in_b"].reshape(1, d),
      lp["s_w1"], lp["s_b1"].reshape(1, d2),
      lp["s_w2"], lp["s_b2"].reshape(1, d),
      lp["ff_ln_w"].reshape(1, d), lp["ff_ln_b"].reshape(1, d),
      lp["ff_w1"], lp["ff_b1"].reshape(1, h4),
      lp["ff_w2"], lp["ff_b2"].reshape(1, d))


def _final_ln_body(x_ref, w_ref, b_ref, out_ref):
    out_ref[...] = _ln(x_ref[...], w_ref[...], b_ref[...])


def _final_ln(x, w, b):
    n, d = x.shape
    tt = n // 4
    return pl.pallas_call(
        _final_ln_body,
        out_shape=jax.ShapeDtypeStruct((n, d), jnp.float32),
        grid=(4,),
        in_specs=[
            pl.BlockSpec((tt, d), lambda t: (t, 0)),
            pl.BlockSpec((1, d), lambda t: (0, 0)),
            pl.BlockSpec((1, d), lambda t: (0, 0)),
        ],
        out_specs=pl.BlockSpec((tt, d), lambda t: (t, 0)),
        compiler_params=pltpu.CompilerParams(
            dimension_semantics=("parallel",)),
        name="final_ln",
    )(x, w.reshape(1, d), b.reshape(1, d))


_HT = 512


def _head_body(h_ref, w_ref, out_ref):
    out_ref[0] = jnp.dot(h_ref[...], w_ref[...],
                         preferred_element_type=jnp.float32)


def _head(h, w, b, t):
    n, d = h.shape
    v = w.shape[1]
    tpb = t // _HT
    return pl.pallas_call(
        _head_body,
        out_shape=jax.ShapeDtypeStruct((b, t, v), jnp.float32),
        grid=(v // _VT, n // _HT),
        in_specs=[
            pl.BlockSpec((_HT, d), lambda vi, ti: (ti, 0)),
            pl.BlockSpec((d, _VT), lambda vi, ti: (0, vi)),
        ],
        out_specs=pl.BlockSpec((1, _HT, _VT),
                               lambda vi, ti: (ti // tpb, ti % tpb, vi)),
        compiler_params=pltpu.CompilerParams(
            dimension_semantics=("parallel", "arbitrary"),
            vmem_limit_bytes=50 * 1024 * 1024),
        name="lm_head",
    )(h, w)


def _head_sel_body(h_ref, n0_ref, n1_ref, s_ref, w_ref, out_ref):
    sel = s_ref[0, 0] <= s_ref[0, 1]
    nz = jnp.where(sel, n0_ref[...], n1_ref[...])
    out_ref[0] = jnp.dot(h_ref[...] + nz, w_ref[...],
                         preferred_element_type=jnp.float32)


def _head_sel(h, n0, n1, s, w, b, t):
    n, d = h.shape
    v = w.shape[1]
    tpb = t // _HT
    return pl.pallas_call(
        _head_sel_body,
        out_shape=jax.ShapeDtypeStruct((b, t, v), jnp.float32),
        grid=(v // _VT, n // _HT),
        in_specs=[
            pl.BlockSpec((_HT, d), lambda vi, ti: (ti, 0)),
            pl.BlockSpec((_HT, d), lambda vi, ti: (ti, 0)),
            pl.BlockSpec((_HT, d), lambda vi, ti: (ti, 0)),
            pl.BlockSpec((1, 4), lambda vi, ti: (0, 0)),
            pl.BlockSpec((d, _VT), lambda vi, ti: (0, vi)),
        ],
        out_specs=pl.BlockSpec((1, _HT, _VT),
                               lambda vi, ti: (ti // tpb, ti % tpb, vi)),
        compiler_params=pltpu.CompilerParams(
            dimension_semantics=("parallel", "arbitrary"),
            vmem_limit_bytes=50 * 1024 * 1024),
        name="lm_head_select",
    )(h, n0, n1, s, w)


def _vote_stats_body(h_ref, n0_ref, n1_ref, hl_ref, w_ref,
                     st_ref, hl_out_ref):
    ti = pl.program_id(1)
    hb = h_ref[...]
    wb = w_ref[...]

    @pl.when(ti == 0)
    def _():
        hl_out_ref[0] = jnp.dot(hl_ref[...], wb,
                                preferred_element_type=jnp.float32)

    cols = []
    for nz in (n0_ref, n1_ref):
        c = jnp.dot(hb + nz[...], wb, preferred_element_type=jnp.float32)
        m = jnp.max(c, axis=-1, keepdims=True)
        ez = jnp.exp(c - m)
        z = jnp.sum(ez, axis=-1, keepdims=True)
        sx = jnp.sum(c * ez, axis=-1, keepdims=True)
        cols += [m, z, sx]
    st_ref[0] = jnp.concatenate(cols, axis=-1)


def _vote_stats(h, n0, n1, hl, w):
    n, d = h.shape
    v = w.shape[1]
    nv = v // _VT
    nt = n // _HT
    return pl.pallas_call(
        _vote_stats_body,
        out_shape=(
            jax.ShapeDtypeStruct((nv, n, 6), jnp.float32),
            jax.ShapeDtypeStruct((nv, hl.shape[0], _VT), jnp.float32),
        ),
        grid=(nv, nt),
        in_specs=[
            pl.BlockSpec((_HT, d), lambda vi, ti: (ti, 0)),
            pl.BlockSpec((_HT, d), lambda vi, ti: (ti, 0)),
            pl.BlockSpec((_HT, d), lambda vi, ti: (ti, 0)),
            pl.BlockSpec((hl.shape[0], d), lambda vi, ti: (0, 0)),
            pl.BlockSpec((d, _VT), lambda vi, ti: (0, vi)),
        ],
        out_specs=(
            pl.BlockSpec((1, _HT, 6), lambda vi, ti: (vi, ti, 0)),
            pl.BlockSpec((1, hl.shape[0], _VT), lambda vi, ti: (vi, 0, 0)),
        ),
        compiler_params=pltpu.CompilerParams(
            dimension_semantics=("parallel", "arbitrary"),
            vmem_limit_bytes=50 * 1024 * 1024),
        name="vote_entropy_stats",
    )(h, n0, n1, hl, w)


def _vote_merge_body(st_ref, hl_ref, out_ref):
    hts = []
    for i in range(2):
        m_v = st_ref[3 * i + 0]
        z_v = st_ref[3 * i + 1]
        sx_v = st_ref[3 * i + 2]
        m = jnp.max(m_v, axis=0)
        scale = jnp.exp(m_v - m[None])
        zz = jnp.sum(z_v * scale, axis=0)
        sxx = jnp.sum(sx_v * scale, axis=0)
        hts.append(m + jnp.log(zz) - sxx / zz)
    base = hts[0][0:1]
    res = [(base + jnp.mean(ht - base)).reshape(1, 1) for ht in hts]
    full = hl_ref[...]
    m = jnp.max(jnp.max(full, axis=0), axis=-1, keepdims=True)
    p = jnp.exp(full - m[None])
    z = jnp.sum(jnp.sum(p, axis=0), axis=-1, keepdims=True)
    pn = p / z[None]
    term = pn * jnp.log(jnp.maximum(pn, 1e-9))
    htr = -jnp.sum(jnp.sum(term, axis=0), axis=-1)
    res.append(jnp.mean(htr).reshape(1, 1))
    res.append(jnp.zeros((1, 1), jnp.float32))
    out_ref[...] = jnp.concatenate(res, axis=1)


def _vote_merge(st_t, hl):
    return pl.pallas_call(
        _vote_merge_body,
        out_shape=jax.ShapeDtypeStruct((1, 4), jnp.float32),
        name="vote_entropy_merge",
    )(st_t, hl)


def kernel(input_ids, params):
    b, t = input_ids.shape
    tok = params["tok"]
    d = tok.shape[1]
    n = b * t
    ids = input_ids.reshape(n).astype(jnp.int32)

    x = _embed(ids, tok, params["pos"][:t])
    for lp in params["layers"]:
        moe = _moe(x, lp)
        x = _ffn(x, moe, lp)
    h = _final_ln(x, params["norm_w"], params["norm_b"])

    w = params["lm_head"]
    h_bt = h.reshape(b, t, d)
    h_last = h_bt[:, -1, :]

    with jax.ensure_compile_time_eval():
        nk = jax.random.split(jax.random.key(1234), 2)
        n0 = (0.01 * jax.random.normal(nk[0], (b, t, d), jnp.float32)
              ).reshape(n, d)
        n1 = (0.01 * jax.random.normal(nk[1], (b, t, d), jnp.float32)
              ).reshape(n, d)

    st, hl_log = _vote_stats(h, n0, n1, h_last, w)
    s = _vote_merge(st.transpose(2, 0, 1), hl_log)

    logits = lax.cond(
        s[0, 2] >= _ENTROPY_TRIG,
        lambda _: _head_sel(h, n0, n1, s, w, b, t),
        lambda _: _head(h, w, b, t),
        None)
    return logits


# speedup vs baseline: 1.0108x; 1.0108x over previous
import jax
import jax.numpy as jnp
from jax import lax
from jax.experimental import pallas as pl
from jax.experimental.pallas import tpu as pltpu

_TEMP = 0.7
_ENTROPY_TRIG = 2.2
_EPS = 1e-5
_E = 16
_VT = 3200


def _ln(x, w, b):
    mu = jnp.mean(x, axis=-1, keepdims=True)
    var = jnp.mean((x - mu) ** 2, axis=-1, keepdims=True)
    return (x - mu) / jnp.sqrt(var + _EPS) * w + b


_EMB_TILE = 256


def _embed_body(ids_ref, tok_hbm, pos_ref, out_ref, sem):
    i = pl.program_id(0)
    base = i * _EMB_TILE
    copies = []
    for mi in range(_EMB_TILE):
        c = pltpu.make_async_copy(
            tok_hbm.at[pl.ds(ids_ref[base + mi], 1), :], out_ref.at[mi], sem)
        c.start()
        copies.append(c)
    for c in copies:
        c.wait()
    out_ref[...] = out_ref[...] + pos_ref[...]


def _embed(ids, tok, pos):
    n = ids.shape[0]
    t = pos.shape[0]
    d = tok.shape[1]
    pos3 = pos.reshape(t, 1, d)
    n_tiles = n // _EMB_TILE
    pos_tiles = t // _EMB_TILE
    out = pl.pallas_call(
        _embed_body,
        out_shape=jax.ShapeDtypeStruct((n, 1, d), jnp.float32),
        grid_spec=pltpu.PrefetchScalarGridSpec(
            num_scalar_prefetch=1,
            grid=(n_tiles,),
            in_specs=[
                pl.BlockSpec(memory_space=pl.ANY),
                pl.BlockSpec((_EMB_TILE, 1, d),
                             lambda i, ids_r: (i % pos_tiles, 0, 0)),
            ],
            out_specs=pl.BlockSpec((_EMB_TILE, 1, d),
                                   lambda i, ids_r: (i, 0, 0)),
            scratch_shapes=[pltpu.SemaphoreType.DMA],
        ),
        compiler_params=pltpu.CompilerParams(
            dimension_semantics=("arbitrary",)),
        name="embed_gather",
    )(ids, tok, pos3)
    return out.reshape(n, d)


def _moe_body(x_ref, lnw_ref, lnb_ref, rw_ref, rb_ref,
              w1_ref, b1_ref, w2_ref, b2_ref,
              moe_ref, hn_ref, wf_ref):
    e = pl.program_id(1)

    @pl.when(e == 0)
    def _():
        x = x_ref[...]
        hn = _ln(x, lnw_ref[...], lnb_ref[...])
        hn_ref[...] = hn
        logits = jnp.dot(hn, rw_ref[...],
                         preferred_element_type=jnp.float32) + rb_ref[...]
        g = jax.nn.softmax(logits / _TEMP, axis=-1)
        iota = lax.broadcasted_iota(jnp.int32, g.shape, 1)
        m1 = jnp.max(g, axis=-1, keepdims=True)
        a1 = jnp.min(jnp.where(g == m1, iota, _E), axis=-1, keepdims=True)
        g2 = jnp.where(iota == a1, -jnp.inf, g)
        m2 = jnp.max(g2, axis=-1, keepdims=True)
        a2 = jnp.min(jnp.where(g2 == m2, iota, _E), axis=-1, keepdims=True)
        wf_ref[...] = (jnp.where(iota == a1, m1, 0.0)
                       + jnp.where(iota == a2, m2, 0.0))
        moe_ref[...] = jnp.zeros_like(moe_ref)

    hn = hn_ref[...]
    mid = jax.nn.silu(
        jnp.dot(hn, w1_ref[0], preferred_element_type=jnp.float32)
        + b1_ref[0])
    eo = (jnp.dot(mid, w2_ref[0], preferred_element_type=jnp.float32)
          + b2_ref[0])
    iota = lax.broadcasted_iota(jnp.int32, wf_ref.shape, 1)
    wcol = jnp.sum(wf_ref[...] * (iota == e).astype(jnp.float32),
                   axis=-1, keepdims=True)
    moe_ref[...] = moe_ref[...] + wcol * eo


def _moe(x, lp):
    n, d = x.shape
    h4 = lp["e_w1"].shape[2]
    tt = n // 2
    return pl.pallas_call(
        _moe_body,
        out_shape=jax.ShapeDtypeStruct((n, d), jnp.float32),
        grid=(2, _E),
        in_specs=[
            pl.BlockSpec((tt, d), lambda t, e: (t, 0)),
            pl.BlockSpec((1, d), lambda t, e: (0, 0)),
            pl.BlockSpec((1, d), lambda t, e: (0, 0)),
            pl.BlockSpec((d, _E), lambda t, e: (0, 0)),
            pl.BlockSpec((1, _E), lambda t, e: (0, 0)),
            pl.BlockSpec((1, d, h4), lambda t, e: (e, 0, 0)),
            pl.BlockSpec((1, 1, h4), lambda t, e: (e, 0, 0)),
            pl.BlockSpec((1, h4, d), lambda t, e: (e, 0, 0)),
            pl.BlockSpec((1, 1, d), lambda t, e: (e, 0, 0)),
        ],
        out_specs=pl.BlockSpec((tt, d), lambda t, e: (t, 0)),
        scratch_shapes=[
            pltpu.VMEM((tt, d), jnp.float32),
            pltpu.VMEM((tt, _E), jnp.float32),
        ],
        compiler_params=pltpu.CompilerParams(
            dimension_semantics=("parallel", "arbitrary"),
            vmem_limit_bytes=50 * 1024 * 1024,
        ),
        name="moe_experts",
    )(x, lp["ln_in_w"].reshape(1, d), lp["ln_in_b"].reshape(1, d),
      lp["router_w"], lp["router_b"].reshape(1, _E),
      lp["e_w1"], lp["e_b1"].reshape(_E, 1, h4),
      lp["e_w2"], lp["e_b2"].reshape(_E, 1, d))


def _ffn_body(x_ref, moe_ref, lnw_ref, lnb_ref,
              sw1_ref, sb1_ref, sw2_ref, sb2_ref,
              flnw_ref, flnb_ref, fw1_ref, fb1_ref, fw2_ref, fb2_ref,
              out_ref):
    x = x_ref[...]
    hn = _ln(x, lnw_ref[...], lnb_ref[...])
    shared = (jnp.dot(
        jax.nn.silu(jnp.dot(hn, sw1_ref[...],
                            preferred_element_type=jnp.float32)
                    + sb1_ref[...]),
        sw2_ref[...], preferred_element_type=jnp.float32)
        + sb2_ref[...]) * 0.25
    moe = moe_ref[...] + shared
    fh = _ln(moe, flnw_ref[...], flnb_ref[...])
    ff = (jnp.dot(
        jax.nn.silu(jnp.dot(fh, fw1_ref[...],
                            preferred_element_type=jnp.float32)
                    + fb1_ref[...]),
        fw2_ref[...], preferred_element_type=jnp.float32)
        + fb2_ref[...])
    out_ref[...] = x + moe + ff


def _ffn(x, moe, lp):
    n, d = x.shape
    d2 = lp["s_w1"].shape[1]
    h4 = lp["ff_w1"].shape[1]
    tt = n // 4
    return pl.pallas_call(
        _ffn_body,
        out_shape=jax.ShapeDtypeStruct((n, d), jnp.float32),
        grid=(4,),
        in_specs=[
            pl.BlockSpec((tt, d), lambda t: (t, 0)),
            pl.BlockSpec((tt, d), lambda t: (t, 0)),
            pl.BlockSpec((1, d), lambda t: (0, 0)),
            pl.BlockSpec((1, d), lambda t: (0, 0)),
            pl.BlockSpec((d, d2), lambda t: (0, 0)),
            pl.BlockSpec((1, d2), lambda t: (0, 0)),
            pl.BlockSpec((d2, d), lambda t: (0, 0)),
            pl.BlockSpec((1, d), lambda t: (0, 0)),
            pl.BlockSpec((1, d), lambda t: (0, 0)),
            pl.BlockSpec((1, d), lambda t: (0, 0)),
            pl.BlockSpec((d, h4), lambda t: (0, 0)),
            pl.BlockSpec((1, h4), lambda t: (0, 0)),
            pl.BlockSpec((h4, d), lambda t: (0, 0)),
            pl.BlockSpec((1, d), lambda t: (0, 0)),
        ],
        out_specs=pl.BlockSpec((tt, d), lambda t: (t, 0)),
        compiler_params=pltpu.CompilerParams(
            dimension_semantics=("parallel",),
            vmem_limit_bytes=50 * 1024 * 1024,
        ),
        name="shared_ffn",
    )(x, moe, lp["ln_in_w"].reshape(1, d), lp["ln_in_b"].reshape(1, d),
      lp["s_w1"], lp["s_b1"].reshape(1, d2),
      lp["s_w2"], lp["s_b2"].reshape(1, d),
      lp["ff_ln_w"].reshape(1, d), lp["ff_ln_b"].reshape(1, d),
      lp["ff_w1"], lp["ff_b1"].reshape(1, h4),
      lp["ff_w2"], lp["ff_b2"].reshape(1, d))


def _final_ln_body(x_ref, w_ref, b_ref, out_ref):
    out_ref[...] = _ln(x_ref[...], w_ref[...], b_ref[...])


def _final_ln(x, w, b):
    n, d = x.shape
    tt = n // 4
    return pl.pallas_call(
        _final_ln_body,
        out_shape=jax.ShapeDtypeStruct((n, d), jnp.float32),
        grid=(4,),
        in_specs=[
            pl.BlockSpec((tt, d), lambda t: (t, 0)),
            pl.BlockSpec((1, d), lambda t: (0, 0)),
            pl.BlockSpec((1, d), lambda t: (0, 0)),
        ],
        out_specs=pl.BlockSpec((tt, d), lambda t: (t, 0)),
        compiler_params=pltpu.CompilerParams(
            dimension_semantics=("parallel",)),
        name="final_ln",
    )(x, w.reshape(1, d), b.reshape(1, d))


_HT = 512


def _head_body(h_ref, w_ref, out_ref):
    out_ref[0] = jnp.dot(h_ref[...], w_ref[...],
                         preferred_element_type=jnp.float32)


def _head(h, w, b, t):
    n, d = h.shape
    v = w.shape[1]
    tpb = t // _HT
    return pl.pallas_call(
        _head_body,
        out_shape=jax.ShapeDtypeStruct((b, t, v), jnp.float32),
        grid=(v // _VT, n // _HT),
        in_specs=[
            pl.BlockSpec((_HT, d), lambda vi, ti: (ti, 0)),
            pl.BlockSpec((d, _VT), lambda vi, ti: (0, vi)),
        ],
        out_specs=pl.BlockSpec((1, _HT, _VT),
                               lambda vi, ti: (ti // tpb, ti % tpb, vi)),
        compiler_params=pltpu.CompilerParams(
            dimension_semantics=("parallel", "arbitrary"),
            vmem_limit_bytes=50 * 1024 * 1024),
        name="lm_head",
    )(h, w)


def _head_sel_body(h_ref, n0_ref, n1_ref, s_ref, w_ref, out_ref):
    sel = s_ref[0, 0] <= s_ref[0, 1]
    nz = jnp.where(sel, n0_ref[...], n1_ref[...])
    out_ref[0] = jnp.dot(h_ref[...] + nz, w_ref[...],
                         preferred_element_type=jnp.float32)


def _head_sel(h, n0, n1, s, w, b, t):
    n, d = h.shape
    v = w.shape[1]
    tpb = t // _HT
    return pl.pallas_call(
        _head_sel_body,
        out_shape=jax.ShapeDtypeStruct((b, t, v), jnp.float32),
        grid=(v // _VT, n // _HT),
        in_specs=[
            pl.BlockSpec((_HT, d), lambda vi, ti: (ti, 0)),
            pl.BlockSpec((_HT, d), lambda vi, ti: (ti, 0)),
            pl.BlockSpec((_HT, d), lambda vi, ti: (ti, 0)),
            pl.BlockSpec((1, 4), lambda vi, ti: (0, 0)),
            pl.BlockSpec((d, _VT), lambda vi, ti: (0, vi)),
        ],
        out_specs=pl.BlockSpec((1, _HT, _VT),
                               lambda vi, ti: (ti // tpb, ti % tpb, vi)),
        compiler_params=pltpu.CompilerParams(
            dimension_semantics=("parallel", "arbitrary"),
            vmem_limit_bytes=50 * 1024 * 1024),
        name="lm_head_select",
    )(h, n0, n1, s, w)


def _vote_stats_body(h_ref, n0_ref, n1_ref, hl_ref, w_ref,
                     st_ref, hl_out_ref):
    ti = pl.program_id(1)
    hb = h_ref[...]
    wb = w_ref[...]

    @pl.when(ti == 0)
    def _():
        hl_out_ref[0] = jnp.dot(hl_ref[...], wb,
                                preferred_element_type=jnp.float32)

    cols = []
    for nz in (n0_ref, n1_ref):
        c = jnp.dot(hb + nz[...], wb, preferred_element_type=jnp.float32)
        m = jnp.max(c, axis=-1, keepdims=True)
        ez = jnp.exp(c - m)
        z = jnp.sum(ez, axis=-1, keepdims=True)
        sx = jnp.sum(c * ez, axis=-1, keepdims=True)
        cols += [m, z, sx]
    st_ref[0] = jnp.concatenate(cols, axis=-1)


def _vote_stats(h, n0, n1, hl, w):
    n, d = h.shape
    v = w.shape[1]
    nv = v // _VT
    nt = n // _HT
    return pl.pallas_call(
        _vote_stats_body,
        out_shape=(
            jax.ShapeDtypeStruct((nv, n, 6), jnp.float32),
            jax.ShapeDtypeStruct((nv, hl.shape[0], _VT), jnp.float32),
        ),
        grid=(nv, nt),
        in_specs=[
            pl.BlockSpec((_HT, d), lambda vi, ti: (ti, 0)),
            pl.BlockSpec((_HT, d), lambda vi, ti: (ti, 0)),
            pl.BlockSpec((_HT, d), lambda vi, ti: (ti, 0)),
            pl.BlockSpec((hl.shape[0], d), lambda vi, ti: (0, 0)),
            pl.BlockSpec((d, _VT), lambda vi, ti: (0, vi)),
        ],
        out_specs=(
            pl.BlockSpec((1, _HT, 6), lambda vi, ti: (vi, ti, 0)),
            pl.BlockSpec((1, hl.shape[0], _VT), lambda vi, ti: (vi, 0, 0)),
        ),
        compiler_params=pltpu.CompilerParams(
            dimension_semantics=("parallel", "arbitrary"),
            vmem_limit_bytes=50 * 1024 * 1024),
        name="vote_entropy_stats",
    )(h, n0, n1, hl, w)


def _vote_merge_body(st_ref, hl_ref, out_ref):
    hts = []
    for i in range(2):
        m_v = st_ref[3 * i + 0]
        z_v = st_ref[3 * i + 1]
        sx_v = st_ref[3 * i + 2]
        m = jnp.max(m_v, axis=0)
        scale = jnp.exp(m_v - m[None])
        zz = jnp.sum(z_v * scale, axis=0)
        sxx = jnp.sum(sx_v * scale, axis=0)
        hts.append(m + jnp.log(zz) - sxx / zz)
    base = hts[0][0:1]
    res = [(base + jnp.mean(ht - base)).reshape(1, 1) for ht in hts]
    full = hl_ref[...]
    m = jnp.max(jnp.max(full, axis=0), axis=-1, keepdims=True)
    p = jnp.exp(full - m[None])
    z = jnp.sum(jnp.sum(p, axis=0), axis=-1, keepdims=True)
    pn = p / z[None]
    term = pn * jnp.log(jnp.maximum(pn, 1e-9))
    htr = -jnp.sum(jnp.sum(term, axis=0), axis=-1)
    res.append(jnp.mean(htr).reshape(1, 1))
    res.append(jnp.zeros((1, 1), jnp.float32))
    out_ref[...] = jnp.concatenate(res, axis=1)


def _vote_merge(st_t, hl):
    return pl.pallas_call(
        _vote_merge_body,
        out_shape=jax.ShapeDtypeStruct((1, 4), jnp.float32),
        name="vote_entropy_merge",
    )(st_t, hl)


def kernel(input_ids, params):
    b, t = input_ids.shape
    tok = params["tok"]
    d = tok.shape[1]
    n = b * t
    ids = input_ids.reshape(n).astype(jnp.int32)

    x = _embed(ids, tok, params["pos"][:t])
    for lp in params["layers"]:
        moe = _moe(x, lp)
        x = _ffn(x, moe, lp)
    h = _final_ln(x, params["norm_w"], params["norm_b"])

    w = params["lm_head"]
    h_bt = h.reshape(b, t, d)
    h_last = h_bt[:, -1, :]

    with jax.ensure_compile_time_eval():
        nk = jax.random.split(jax.random.key(1234), 2)
        n0 = (0.01 * jax.random.normal(nk[0], (b, t, d), jnp.float32)
              ).reshape(n, d)
        n1 = (0.01 * jax.random.normal(nk[1], (b, t, d), jnp.float32)
              ).reshape(n, d)

    st, hl_log = _vote_stats(h, n0, n1, h_last, w)
    s = _vote_merge(st.transpose(2, 0, 1), hl_log)

    logits = lax.cond(
        s[0, 2] >= _ENTROPY_TRIG,
        lambda _: _head_sel(h, n0, n1, s, w, b, t),
        lambda _: _head(h, w, b, t),
        None)
    return logits
